# baseline (speedup 1.0000x reference)
import functools

import jax
import jax.numpy as jnp
from jax import lax
from jax.experimental import pallas as pl
from jax.experimental.pallas import tpu as pltpu
from jax.experimental.pallas import tpu_sc as plsc

NC = 2
NS = 16
L = 16
K = 64
IB = 8


def _make_sc_aggregate(n_pad, d, cpw):
    dh = d // 2
    np2 = n_pad // 2
    rpt = np2 // NS
    nblocks = cpw // IB

    mesh = plsc.VectorSubcoreMesh(
        core_axis_name="c", subcore_axis_name="s",
        num_cores=NC, num_subcores=NS)

    @functools.partial(
        pl.kernel,
        out_type=jax.ShapeDtypeStruct((NC, np2, d), jnp.float32),
        mesh=mesh,
        scratch_types=[
            pltpu.VMEM((2, IB, K), jnp.int32),
            pltpu.VMEM((2, IB, K), jnp.int32),
            pltpu.VMEM((2, IB, K), jnp.float32),
            pltpu.VMEM((K,), jnp.int32),
            pltpu.VMEM((K,), jnp.int32),
            pltpu.VMEM((2, K, d), jnp.float32),
            pltpu.VMEM((K, d), jnp.float32),
            pltpu.VMEM_SHARED((np2, d), jnp.float32),
            pltpu.VMEM_SHARED((np2, d), jnp.float32),
            pltpu.SemaphoreType.DMA((2,)),
            pltpu.SemaphoreType.DMA,
            pltpu.SemaphoreType.DMA,
        ],
    )
    def sc_aggregate(src_hbm, dst_hbm, e_hbm, x_hbm, dummy_hbm, out_hbm,
                     src_v, dst_v, e_v, gs_v, gd_v, rows_v, stag,
                     x_sp, acc, gsem, ssem, isem):
        cid = lax.axis_index("c")
        sid = lax.axis_index("s")
        col0 = cid * dh
        pbase = sid * rpt
        zeros16 = jnp.zeros((L,), jnp.float32)

        def zrow(r, carry):
            for k2 in range(d // L):
                rows_v[0, r, pl.ds(k2 * L, L)] = zeros16
            return carry
        lax.fori_loop(0, K, zrow, 0)
        for j in range(rpt // K):
            pltpu.sync_copy(rows_v.at[0],
                            acc.at[pl.ds(pbase + j * K, K)])

        for j in range(rpt // 32):
            lbase = 2 * pbase + j * 64
            pltpu.sync_copy(x_hbm.at[pl.ds(lbase, 64)], stag)

            def pack_in(pr, carry):
                for k2 in range(dh // L):
                    rows_v[0, pr, pl.ds(k2 * L, L)] = (
                        stag[2 * pr, pl.ds(col0 + k2 * L, L)])
                    rows_v[0, pr, pl.ds(dh + k2 * L, L)] = (
                        stag[2 * pr + 1, pl.ds(col0 + k2 * L, L)])
                return carry
            lax.fori_loop(0, 32, pack_in, 0)
            pltpu.sync_copy(rows_v.at[0, pl.ds(0, 32)],
                            x_sp.at[pl.ds(pbase + j * 32, 32)])
        plsc.subcore_barrier()

        def load_idx_block(bo, ib):
            pltpu.async_copy(src_hbm.at[sid, pl.ds(bo * IB, IB)],
                             src_v.at[ib], isem)
            pltpu.async_copy(dst_hbm.at[sid, pl.ds(bo * IB, IB)],
                             dst_v.at[ib], isem)
            pltpu.async_copy(e_hbm.at[sid, pl.ds(bo * IB, IB)],
                             e_v.at[ib], isem)

        def wait_idx_block():
            for _ in range(2):
                pltpu.make_async_copy(src_hbm.at[0, pl.ds(0, IB)],
                                      src_v.at[0], isem).wait()
            pltpu.make_async_copy(e_hbm.at[0, pl.ds(0, IB)],
                                  e_v.at[0], isem).wait()

        def fill_gs(ib, h):
            for g in range(K // L):
                gs_v[pl.ds(g * L, L)] = (
                    src_v[ib, h, pl.ds(g * L, L)] >> 1)

        def start_gather(b):
            pltpu.async_copy(x_sp.at[gs_v], rows_v.at[b], gsem.at[b])

        def wait_gather(b):
            pltpu.make_async_copy(dummy_hbm, rows_v.at[b],
                                  gsem.at[b]).wait()

        def start_scatter():
            pltpu.async_copy(stag, acc.at[gd_v], ssem, add=True)

        def wait_scatter():
            pltpu.make_async_copy(dummy_hbm, stag, ssem).wait()

        load_idx_block(0, 0)
        wait_idx_block()
        fill_gs(0, 0)
        start_gather(0)

        def block_body(bo, carry):
            ib = lax.rem(bo, 2)
            nib = lax.rem(bo + 1, 2)
            have_next = bo + 1 < nblocks

            for h in range(IB):
                b = h % 2
                nb = (h + 1) % 2

                wait_gather(b)

                if h == 0:
                    @pl.when(have_next)
                    def _():
                        load_idx_block(bo + 1, nib)

                if h < IB - 1:
                    fill_gs(ib, h + 1)
                    start_gather(nb)
                else:
                    @pl.when(have_next)
                    def _():
                        wait_idx_block()
                        fill_gs(nib, 0)
                        start_gather(nb)

                if h == 0:
                    @pl.when(bo >= 1)
                    def _():
                        wait_scatter()
                else:
                    wait_scatter()

                for g in range(K // L):
                    gd_v[pl.ds(g * L, L)] = (
                        dst_v[ib, h, pl.ds(g * L, L)] >> 1)

                def scale_grp(g, c2):
                    e_vec = e_v[ib, h, pl.ds(g * L, L)]
                    s_vec = src_v[ib, h, pl.ds(g * L, L)]
                    d_vec = dst_v[ib, h, pl.ds(g * L, L)]
                    for i in range(L):
                        r = g * L + i
                        ev = e_vec[i]
                        so = (s_vec[i] & 1) * dh
                        do = (d_vec[i] & 1) * dh
                        zo = dh - do
                        for k2 in range(dh // L):
                            stag[r, pl.ds(do + k2 * L, L)] = (
                                rows_v[b, r, pl.ds(so + k2 * L, L)] * ev)
                            stag[r, pl.ds(zo + k2 * L, L)] = zeros16
                    return c2
                lax.fori_loop(0, K // L, scale_grp, 0)

                start_scatter()
            return carry
        lax.fori_loop(0, nblocks, block_body, 0)

        wait_scatter()
        plsc.subcore_barrier()
        pltpu.sync_copy(acc.at[pl.ds(pbase, rpt)],
                        out_hbm.at[cid, pl.ds(pbase, rpt)])

    return sc_aggregate


def _combine_body(p_ref, w_ref, b_ref, o_ref):
    s = jnp.concatenate([p_ref[0], p_ref[1]], axis=1)
    o_ref[...] = lax.dot_general(
        s, w_ref[...], (((1,), (1,)), ((), ())),
        preferred_element_type=jnp.float32) + b_ref[...]


def kernel(x, edge_index, e, W, b):
    n_nodes, d = x.shape
    e_total = edge_index.shape[1]
    src = edge_index[0].astype(jnp.int32)
    dst = edge_index[1].astype(jnp.int32)
    ef = e[:, 0].astype(jnp.float32)

    cpw = -(-e_total // (NS * K))
    cpw = -(-cpw // IB) * IB
    e_pad = NS * cpw * K
    pad = e_pad - e_total
    if pad:
        spread = (jnp.arange(pad, dtype=jnp.int32) * 2) % n_nodes
        src = jnp.concatenate([src, spread])
        dst = jnp.concatenate([dst, spread])
        ef = jnp.concatenate([ef, jnp.zeros((pad,), jnp.float32)])
    src = src.reshape(NS, cpw, K)
    dst = dst.reshape(NS, cpw, K)
    ef = ef.reshape(NS, cpw, K)

    n_pad = -(-n_nodes // (2 * NS * 64)) * (2 * NS * 64)
    xp = jnp.pad(x, ((0, n_pad - n_nodes), (0, 0)))
    dummy = jnp.zeros((K, d), jnp.float32)
    packed = _make_sc_aggregate(n_pad, d, cpw)(src, dst, ef, xp, dummy)
    partials = packed.reshape(NC, n_pad, d // 2)

    blk = 1000
    grid = n_nodes // blk
    out = pl.pallas_call(
        _combine_body,
        grid=(grid,),
        in_specs=[
            pl.BlockSpec((NC, blk, d // 2), lambda i: (0, i, 0)),
            pl.BlockSpec((d, d), lambda i: (0, 0)),
            pl.BlockSpec((1, d), lambda i: (0, 0)),
        ],
        out_specs=pl.BlockSpec((blk, d), lambda i: (i, 0)),
        out_shape=jax.ShapeDtypeStruct((n_nodes, d), jnp.float32),
    )(partials, W, b.reshape(1, d))
    return out


# speedup vs baseline: 1.2648x; 1.2648x over previous
import functools

import jax
import jax.numpy as jnp
from jax import lax
from jax.experimental import pallas as pl
from jax.experimental.pallas import tpu as pltpu
from jax.experimental.pallas import tpu_sc as plsc

NC = 2
NS = 16
L = 16
K = 128
IB = 8
NW = NC * NS


def _make_sc_aggregate(n_pad, d, cpw):
    rows_per_tile = n_pad // NS
    nblocks = cpw // IB

    mesh = plsc.VectorSubcoreMesh(
        core_axis_name="c", subcore_axis_name="s",
        num_cores=NC, num_subcores=NS)

    @functools.partial(
        pl.kernel,
        out_type=jax.ShapeDtypeStruct((NC, n_pad, d), jnp.float32),
        mesh=mesh,
        scratch_types=[
            pltpu.VMEM((2, IB, K), jnp.int32),
            pltpu.VMEM((2, IB, K), jnp.int32),
            pltpu.VMEM((2, IB, K), jnp.float32),
            pltpu.VMEM((2, K, d), jnp.float32),
            pltpu.VMEM_SHARED((n_pad, d), jnp.float32),
            pltpu.SemaphoreType.DMA((2,)),
            pltpu.SemaphoreType.DMA((2,)),
            pltpu.SemaphoreType.DMA,
        ],
    )
    def sc_aggregate(src_hbm, dst_hbm, e_hbm, x_hbm, out_hbm,
                     src_v, dst_v, e_v, rows_v, acc,
                     gsem, ssem, isem):
        cid = lax.axis_index("c")
        sid = lax.axis_index("s")
        wid = sid * NC + cid

        zeros16 = jnp.zeros((L,), jnp.float32)

        def zrow(r, carry):
            for k2 in range(d // L):
                rows_v[0, r, pl.ds(k2 * L, L)] = zeros16
            return carry
        lax.fori_loop(0, K, zrow, 0)
        tile_base = sid * rows_per_tile
        off = 0
        while off < rows_per_tile:
            sz = min(K, rows_per_tile - off)
            pltpu.sync_copy(rows_v.at[0, pl.ds(0, sz)],
                            acc.at[pl.ds(tile_base + off, sz)])
            off += sz
        plsc.subcore_barrier()

        def load_idx_block(bo, ib):
            pltpu.async_copy(src_hbm.at[wid, pl.ds(bo * IB, IB)],
                             src_v.at[ib], isem)
            pltpu.async_copy(dst_hbm.at[wid, pl.ds(bo * IB, IB)],
                             dst_v.at[ib], isem)
            pltpu.async_copy(e_hbm.at[wid, pl.ds(bo * IB, IB)],
                             e_v.at[ib], isem)

        def wait_idx_block():
            for _ in range(2):
                pltpu.make_async_copy(src_hbm.at[0, pl.ds(0, IB)],
                                      src_v.at[0], isem).wait()
            pltpu.make_async_copy(e_hbm.at[0, pl.ds(0, IB)],
                                  e_v.at[0], isem).wait()

        def start_gather(idx_ref, b):
            pltpu.async_copy(x_hbm.at[idx_ref], rows_v.at[b], gsem.at[b])

        def wait_gather(b):
            pltpu.make_async_copy(x_hbm.at[pl.ds(0, K)], rows_v.at[b],
                                  gsem.at[b]).wait()

        def start_scatter(idx_ref, b):
            pltpu.async_copy(rows_v.at[b], acc.at[idx_ref],
                             ssem.at[b], add=True)

        def wait_scatter(b):
            pltpu.make_async_copy(x_hbm.at[pl.ds(0, K)], rows_v.at[b],
                                  ssem.at[b]).wait()

        load_idx_block(0, 0)
        wait_idx_block()
        start_gather(src_v.at[0, 0], 0)

        def block_body(bo, carry):
            ib = lax.rem(bo, 2)
            nib = lax.rem(bo + 1, 2)
            have_next = bo + 1 < nblocks

            for h in range(IB):
                b = h % 2
                nb = (h + 1) % 2

                if h == 0:
                    @pl.when(bo >= 1)
                    def _():
                        wait_scatter(nb)

                    @pl.when(have_next)
                    def _():
                        load_idx_block(bo + 1, nib)
                else:
                    wait_scatter(nb)

                if h < IB - 1:
                    start_gather(src_v.at[ib, h + 1], nb)
                else:
                    @pl.when(have_next)
                    def _():
                        wait_idx_block()
                        start_gather(src_v.at[nib, 0], nb)

                wait_gather(b)

                def scale_grp(g, c2):
                    e_vec = e_v[ib, h, pl.ds(g * L, L)]
                    for i in range(L):
                        ev = e_vec[i]
                        r = g * L + i
                        for k2 in range(d // L):
                            sl = pl.ds(k2 * L, L)
                            rows_v[b, r, sl] = rows_v[b, r, sl] * ev
                    return c2
                lax.fori_loop(0, K // L, scale_grp, 0)

                start_scatter(dst_v.at[ib, h], b)
            return carry
        lax.fori_loop(0, nblocks, block_body, 0)

        wait_scatter((cpw - 1) % 2)

        plsc.subcore_barrier()
        pltpu.sync_copy(acc.at[pl.ds(tile_base, rows_per_tile)],
                        out_hbm.at[cid, pl.ds(tile_base, rows_per_tile)])

    return sc_aggregate


def _combine_body(p_ref, w_ref, b_ref, o_ref):
    s = p_ref[0] + p_ref[1]
    o_ref[...] = lax.dot_general(
        s, w_ref[...], (((1,), (1,)), ((), ())),
        preferred_element_type=jnp.float32) + b_ref[...]


def kernel(x, edge_index, e, W, b):
    n_nodes, d = x.shape
    e_total = edge_index.shape[1]
    src = edge_index[0].astype(jnp.int32)
    dst = edge_index[1].astype(jnp.int32)
    ef = e[:, 0].astype(jnp.float32)

    cpw = -(-e_total // (NW * K))
    cpw = -(-cpw // IB) * IB
    e_pad = NW * cpw * K
    pad = e_pad - e_total
    if pad:
        src = jnp.pad(src, (0, pad))
        dst = jnp.pad(dst, (0, pad))
        ef = jnp.pad(ef, (0, pad))
    src = src.reshape(NW, cpw, K)
    dst = dst.reshape(NW, cpw, K)
    ef = ef.reshape(NW, cpw, K)

    n_pad = -(-n_nodes // (8 * NS)) * (8 * NS)
    partials = _make_sc_aggregate(n_pad, d, cpw)(src, dst, ef, x)

    blk = 1000
    grid = n_nodes // blk
    out = pl.pallas_call(
        _combine_body,
        grid=(grid,),
        in_specs=[
            pl.BlockSpec((NC, blk, d), lambda i: (0, i, 0)),
            pl.BlockSpec((d, d), lambda i: (0, 0)),
            pl.BlockSpec((1, d), lambda i: (0, 0)),
        ],
        out_specs=pl.BlockSpec((blk, d), lambda i: (i, 0)),
        out_shape=jax.ShapeDtypeStruct((n_nodes, d), jnp.float32),
    )(partials, W, b.reshape(1, d))
    return out
